# baseline (speedup 1.0000x reference)
import functools

import jax
import jax.numpy as jnp
from jax import lax
from jax.experimental import pallas as pl
from jax.experimental.pallas import tpu as pltpu
from jax.experimental.pallas import tpu_sc as plsc

N_NODES = 10000
N_EDGES = 320000
D_FEAT = 128

NC = 2
NS = 16
NW = NC * NS

CHUNK = 128
NCHUNK = 80
E_TILE = CHUNK * NCHUNK
E_PAD = NW * E_TILE

ROWS_TILE = 632
ACC_ROWS = NS * ROWS_TILE

_mesh = plsc.VectorSubcoreMesh(core_axis_name="c", subcore_axis_name="s")


@functools.partial(
    pl.kernel,
    mesh=_mesh,
    out_type=jax.ShapeDtypeStruct((NC, ACC_ROWS, D_FEAT), jnp.float32),
    scratch_types=[
        pltpu.VMEM((NCHUNK // 2, CHUNK), jnp.int32),
        pltpu.VMEM((NCHUNK // 2, CHUNK), jnp.int32),
        pltpu.VMEM((CHUNK, D_FEAT), jnp.float32),
        pltpu.VMEM((CHUNK, D_FEAT), jnp.float32),
        pltpu.VMEM_SHARED((ACC_ROWS, D_FEAT), jnp.float32),
        pltpu.SemaphoreType.DMA,
        pltpu.SemaphoreType.DMA,
    ],
)
def _scatter_partials(x_hbm, src_hbm, dst_hbm, zeros_hbm, out_hbm,
                      src_v, dst_v, buf0, buf1, acc, sem0, sem1):
    c = lax.axis_index("c")
    s = lax.axis_index("s")
    wid = s * NC + c

    pltpu.sync_copy(zeros_hbm, buf0)
    base = s * ROWS_TILE
    for r in range(0, 512, CHUNK):
        pltpu.sync_copy(buf0, acc.at[pl.ds(base + r, CHUNK)])
    tail = ROWS_TILE - 512
    pltpu.sync_copy(buf0.at[pl.ds(0, tail)], acc.at[pl.ds(base + 512, tail)])
    plsc.subcore_barrier()

    def body(j, carry):
        j0 = 2 * j
        g0 = pltpu.async_copy(x_hbm.at[src_v.at[j0]], buf0, sem0)
        g1 = pltpu.async_copy(x_hbm.at[src_v.at[j0 + 1]], buf1, sem1)
        g0.wait()
        pltpu.sync_copy(buf0, acc.at[dst_v.at[j0]], add=True)
        g1.wait()
        pltpu.sync_copy(buf1, acc.at[dst_v.at[j0 + 1]], add=True)
        return carry

    half = NCHUNK // 2
    for h in range(2):
        pltpu.sync_copy(src_hbm.at[wid].at[pl.ds(h * half, half)], src_v)
        pltpu.sync_copy(dst_hbm.at[wid].at[pl.ds(h * half, half)], dst_v)
        lax.fori_loop(0, half // 2, body, 0)
    plsc.subcore_barrier()

    for r in range(0, 512, CHUNK):
        pltpu.sync_copy(acc.at[pl.ds(base + r, CHUNK)], buf0)
        pltpu.sync_copy(buf0, out_hbm.at[c].at[pl.ds(base + r, CHUNK)])
    pltpu.sync_copy(acc.at[pl.ds(base + 512, tail)], buf1.at[pl.ds(0, tail)])
    pltpu.sync_copy(buf1.at[pl.ds(0, tail)],
                    out_hbm.at[c].at[pl.ds(base + 512, tail)])


_BLK = 1000


def _combine_body(scale_ref, x_ref, p_ref, o_ref):
    o_ref[...] = scale_ref[0, 0] * x_ref[...] + p_ref[0] + p_ref[1]


def _combine(scale, x, partials):
    return pl.pallas_call(
        _combine_body,
        grid=(N_NODES // _BLK,),
        in_specs=[
            pl.BlockSpec(memory_space=pltpu.SMEM),
            pl.BlockSpec((_BLK, D_FEAT), lambda i: (i, 0)),
            pl.BlockSpec((NC, _BLK, D_FEAT), lambda i: (0, i, 0)),
        ],
        out_specs=pl.BlockSpec((_BLK, D_FEAT), lambda i: (i, 0)),
        out_shape=jax.ShapeDtypeStruct((N_NODES, D_FEAT), jnp.float32),
    )(scale, x, partials)


@jax.jit
def kernel(x, edge_index, eps):
    src = edge_index[0]
    dst = edge_index[1]
    pad = E_PAD - N_EDGES
    src_p = jnp.concatenate(
        [src, jnp.zeros((pad,), jnp.int32)]).reshape(NW, NCHUNK, CHUNK)
    pad_dst = N_NODES + (jnp.arange(pad, dtype=jnp.int32) % (ACC_ROWS - N_NODES))
    dst_p = jnp.concatenate([dst, pad_dst]).reshape(NW, NCHUNK, CHUNK)
    zeros = jnp.zeros((CHUNK, D_FEAT), jnp.float32)
    partials = _scatter_partials(x, src_p, dst_p, zeros)
    scale = (1.0 + eps).reshape(1, 1).astype(jnp.float32)
    return _combine(scale, x, partials)


# speedup vs baseline: 1.0049x; 1.0049x over previous
import functools

import jax
import jax.numpy as jnp
from jax import lax
from jax.experimental import pallas as pl
from jax.experimental.pallas import tpu as pltpu
from jax.experimental.pallas import tpu_sc as plsc

N_NODES = 10000
N_EDGES = 320000
D_FEAT = 128

NC = 2
NS = 16
NW = NC * NS

CHUNK = 128
NCHUNK = 80
E_TILE = CHUNK * NCHUNK
E_PAD = NW * E_TILE

ROWS_TILE = 632
ACC_ROWS = NS * ROWS_TILE

_mesh = plsc.VectorSubcoreMesh(core_axis_name="c", subcore_axis_name="s")


@functools.partial(
    pl.kernel,
    mesh=_mesh,
    out_type=jax.ShapeDtypeStruct((NC, ACC_ROWS, D_FEAT), jnp.float32),
    scratch_types=[
        pltpu.VMEM((NCHUNK // 2, CHUNK), jnp.int32),
        pltpu.VMEM((NCHUNK // 2, CHUNK), jnp.int32),
        pltpu.VMEM((CHUNK, D_FEAT), jnp.float32),
        pltpu.VMEM((CHUNK, D_FEAT), jnp.float32),
        pltpu.VMEM_SHARED((ACC_ROWS, D_FEAT), jnp.float32),
        pltpu.SemaphoreType.DMA,
        pltpu.SemaphoreType.DMA,
        pltpu.SemaphoreType.DMA,
        pltpu.SemaphoreType.DMA,
    ],
)
def _scatter_partials(x_hbm, src_hbm, dst_hbm, zeros_hbm, out_hbm,
                      src_v, dst_v, buf0, buf1, acc, sem0, sem1, sem2, sem3):
    c = lax.axis_index("c")
    s = lax.axis_index("s")
    wid = s * NC + c

    pltpu.sync_copy(zeros_hbm, buf0)
    base = s * ROWS_TILE
    for r in range(0, 512, CHUNK):
        pltpu.sync_copy(buf0, acc.at[pl.ds(base + r, CHUNK)])
    tail = ROWS_TILE - 512
    pltpu.sync_copy(buf0.at[pl.ds(0, tail)], acc.at[pl.ds(base + 512, tail)])
    plsc.subcore_barrier()

    def body(j, carry):
        j0 = 2 * j
        g0 = pltpu.async_copy(x_hbm.at[src_v.at[j0]], buf0, sem0)
        g1 = pltpu.async_copy(x_hbm.at[src_v.at[j0 + 1]], buf1, sem1)
        g0.wait()
        s0 = pltpu.async_copy(buf0, acc.at[dst_v.at[j0]], sem2, add=True)
        g1.wait()
        s1 = pltpu.async_copy(buf1, acc.at[dst_v.at[j0 + 1]], sem3, add=True)
        s0.wait()
        s1.wait()
        return carry

    half = NCHUNK // 2
    for h in range(2):
        pltpu.sync_copy(src_hbm.at[wid].at[pl.ds(h * half, half)], src_v)
        pltpu.sync_copy(dst_hbm.at[wid].at[pl.ds(h * half, half)], dst_v)
        lax.fori_loop(0, half // 2, body, 0)
    plsc.subcore_barrier()

    for r in range(0, 512, CHUNK):
        pltpu.sync_copy(acc.at[pl.ds(base + r, CHUNK)], buf0)
        pltpu.sync_copy(buf0, out_hbm.at[c].at[pl.ds(base + r, CHUNK)])
    pltpu.sync_copy(acc.at[pl.ds(base + 512, tail)], buf1.at[pl.ds(0, tail)])
    pltpu.sync_copy(buf1.at[pl.ds(0, tail)],
                    out_hbm.at[c].at[pl.ds(base + 512, tail)])


_BLK = 1000


def _combine_body(scale_ref, x_ref, p_ref, o_ref):
    o_ref[...] = scale_ref[0, 0] * x_ref[...] + p_ref[0] + p_ref[1]


def _combine(scale, x, partials):
    return pl.pallas_call(
        _combine_body,
        grid=(N_NODES // _BLK,),
        in_specs=[
            pl.BlockSpec(memory_space=pltpu.SMEM),
            pl.BlockSpec((_BLK, D_FEAT), lambda i: (i, 0)),
            pl.BlockSpec((NC, _BLK, D_FEAT), lambda i: (0, i, 0)),
        ],
        out_specs=pl.BlockSpec((_BLK, D_FEAT), lambda i: (i, 0)),
        out_shape=jax.ShapeDtypeStruct((N_NODES, D_FEAT), jnp.float32),
    )(scale, x, partials)


@jax.jit
def kernel(x, edge_index, eps):
    src = edge_index[0]
    dst = edge_index[1]
    pad = E_PAD - N_EDGES
    src_p = jnp.concatenate(
        [src, jnp.zeros((pad,), jnp.int32)]).reshape(NW, NCHUNK, CHUNK)
    pad_dst = N_NODES + (jnp.arange(pad, dtype=jnp.int32) % (ACC_ROWS - N_NODES))
    dst_p = jnp.concatenate([dst, pad_dst]).reshape(NW, NCHUNK, CHUNK)
    zeros = jnp.zeros((CHUNK, D_FEAT), jnp.float32)
    partials = _scatter_partials(x, src_p, dst_p, zeros)
    scale = (1.0 + eps).reshape(1, 1).astype(jnp.float32)
    return _combine(scale, x, partials)
